# Initial kernel scaffold; baseline (speedup 1.0000x reference)
import jax
import jax.numpy as jnp
from jax.experimental import pallas as pl


def kernel(v, pref_map, suff_map, E, E_pref, E_suff, W0, b0, W1, b1):
    raise NotImplementedError("write your pallas kernel here")


# speedup vs baseline: 3.6302x; 3.6302x over previous
import functools

import jax
import jax.numpy as jnp
from jax import lax
from jax.experimental import pallas as pl
from jax.experimental.pallas import tpu as pltpu
from jax.experimental.pallas import tpu_sc as plsc

_VOCAB = 100000
_EMB = 50
_EMBP = 64
_WIN = 5
_HID = 150
_TAGS = 45
_BATCH = 16384
_TOK = _BATCH * _WIN
_C = 128


def _gather_sum_sc(v_flat, pref_map, suff_map, e_pad, ep_pad, es_pad):
    info = plsc.get_sparse_core_info()
    nc, ns = info.num_cores, info.num_subcores
    nw = nc * ns
    per_w = _TOK // nw
    n_chunks = per_w // _C
    mesh = plsc.VectorSubcoreMesh(core_axis_name="c", subcore_axis_name="s")

    @functools.partial(
        pl.kernel,
        mesh=mesh,
        compiler_params=pltpu.CompilerParams(use_tc_tiling_on_sc=False),
        out_type=jax.ShapeDtypeStruct((_TOK, _EMBP), jnp.float32),
        scratch_types=[
            pltpu.VMEM((_C,), jnp.int32),
            pltpu.VMEM((_C,), jnp.int32),
            pltpu.VMEM((_C,), jnp.int32),
            pltpu.VMEM((_C, _EMBP), jnp.float32),
            pltpu.VMEM((_C, _EMBP), jnp.float32),
            pltpu.VMEM((_C, _EMBP), jnp.float32),
            pltpu.VMEM((_C, _EMBP), jnp.float32),
            pltpu.SemaphoreType.DMA,
            pltpu.SemaphoreType.DMA,
        ],
    )
    def gather_kernel(v_hbm, pm_hbm, sm_hbm, e_hbm, ep_hbm, es_hbm, out_hbm,
                      vidx, pidx, sidx, buf_e, buf_p, buf_s, buf_o,
                      sem_idx, sem_row):
        wid = lax.axis_index("s") * nc + lax.axis_index("c")
        wbase = wid * per_w

        def chunk_body(ci, carry):
            base = wbase + ci * _C
            pltpu.sync_copy(v_hbm.at[pl.ds(base, _C)], vidx)
            cp_pi = pltpu.async_copy(pm_hbm.at[vidx], pidx, sem_idx)
            cp_si = pltpu.async_copy(sm_hbm.at[vidx], sidx, sem_idx)
            cp_e = pltpu.async_copy(e_hbm.at[vidx], buf_e, sem_row)
            cp_pi.wait()
            cp_si.wait()
            cp_p = pltpu.async_copy(ep_hbm.at[pidx], buf_p, sem_row)
            cp_s = pltpu.async_copy(es_hbm.at[sidx], buf_s, sem_row)
            cp_e.wait()
            cp_p.wait()
            cp_s.wait()

            def row_body(i, c2):
                for o in (0, 16, 32, 48):
                    buf_o[i, pl.ds(o, 16)] = (
                        buf_e[i, pl.ds(o, 16)]
                        + buf_p[i, pl.ds(o, 16)]
                        + buf_s[i, pl.ds(o, 16)]
                    )
                return c2

            lax.fori_loop(0, _C, row_body, 0)
            pltpu.sync_copy(buf_o, out_hbm.at[pl.ds(base, _C)])
            return carry

        lax.fori_loop(0, n_chunks, chunk_body, 0)

    return gather_kernel(v_flat, pref_map, suff_map, e_pad, ep_pad, es_pad)


def _mlp_tc(h, w0, b0, w1, b1):
    bblk = 1024
    kdim = _WIN * _EMBP

    def body(h_ref, w0_ref, b0_ref, w1_ref, b1_ref, o_ref):
        hb = h_ref[...]
        z = jnp.tanh(
            jnp.dot(hb, w0_ref[...], preferred_element_type=jnp.float32)
            + b0_ref[...]
        )
        logits = (
            jnp.dot(z, w1_ref[...], preferred_element_type=jnp.float32)
            + b1_ref[...]
        )
        m = jnp.max(logits, axis=1, keepdims=True)
        s = jnp.sum(jnp.exp(logits - m), axis=1, keepdims=True)
        o_ref[...] = logits - (m + jnp.log(s))

    return pl.pallas_call(
        body,
        grid=(_BATCH // bblk,),
        in_specs=[
            pl.BlockSpec((bblk, kdim), lambda i: (i, 0)),
            pl.BlockSpec((kdim, _HID), lambda i: (0, 0)),
            pl.BlockSpec((1, _HID), lambda i: (0, 0)),
            pl.BlockSpec((_HID, _TAGS), lambda i: (0, 0)),
            pl.BlockSpec((1, _TAGS), lambda i: (0, 0)),
        ],
        out_specs=pl.BlockSpec((bblk, _TAGS), lambda i: (i, 0)),
        out_shape=jax.ShapeDtypeStruct((_BATCH, _TAGS), jnp.float32),
    )(h, w0, b0, w1, b1)


def kernel(v, pref_map, suff_map, E, E_pref, E_suff, W0, b0, W1, b1):
    pad = ((0, 0), (0, _EMBP - _EMB))
    e_pad = jnp.pad(E, pad)
    ep_pad = jnp.pad(E_pref, pad)
    es_pad = jnp.pad(E_suff, pad)
    emb = _gather_sum_sc(v.reshape(-1), pref_map, suff_map,
                         e_pad, ep_pad, es_pad)
    h = emb.reshape(_BATCH, _WIN * _EMBP)
    w0p = jnp.pad(W0.reshape(_WIN, _EMB, _HID),
                  ((0, 0), (0, _EMBP - _EMB), (0, 0))).reshape(
                      _WIN * _EMBP, _HID)
    return _mlp_tc(h, w0p, b0.reshape(1, _HID), W1, b1.reshape(1, _TAGS))


# speedup vs baseline: 3.8885x; 1.0712x over previous
import functools

import jax
import jax.numpy as jnp
from jax import lax
from jax.experimental import pallas as pl
from jax.experimental.pallas import tpu as pltpu
from jax.experimental.pallas import tpu_sc as plsc

_VOCAB = 100000
_EMB = 50
_EMBP = 64
_WIN = 5
_HID = 150
_TAGS = 45
_BATCH = 16384
_C = 128
_NB = _BATCH // _C
_NCHUNKS = _WIN * _NB


def _gather_sum_sc(vt, pref_map, suff_map, e_pad, ep_pad, es_pad):
    info = plsc.get_sparse_core_info()
    nc, ns = info.num_cores, info.num_subcores
    nw = nc * ns
    per_w = _NCHUNKS // nw
    pairs = per_w // 2
    mesh = plsc.VectorSubcoreMesh(core_axis_name="c", subcore_axis_name="s")

    idx_t = pltpu.VMEM((_C,), jnp.int32)
    row_t = pltpu.VMEM((_C, _EMBP), jnp.float32)

    @functools.partial(
        pl.kernel,
        mesh=mesh,
        compiler_params=pltpu.CompilerParams(use_tc_tiling_on_sc=False),
        out_type=jax.ShapeDtypeStruct((_WIN, _BATCH, _EMBP), jnp.float32),
        scratch_types=[
            idx_t, idx_t, idx_t, idx_t, idx_t, idx_t,
            row_t, row_t, row_t, row_t, row_t, row_t,
            row_t, row_t,
            pltpu.SemaphoreType.DMA, pltpu.SemaphoreType.DMA,
            pltpu.SemaphoreType.DMA, pltpu.SemaphoreType.DMA,
            pltpu.SemaphoreType.DMA, pltpu.SemaphoreType.DMA,
        ],
    )
    def gather_kernel(vt_hbm, pm_hbm, sm_hbm, e_hbm, ep_hbm, es_hbm, out_hbm,
                      vi_a, vi_b, pi_a, pi_b, si_a, si_b,
                      be_a, bp_a, bs_a, be_b, bp_b, bs_b,
                      ob_a, ob_b,
                      sem_ia, sem_ib, sem_ra, sem_rb, sem_wa, sem_wb):
        wid = lax.axis_index("s") * nc + lax.axis_index("c")
        cbase = wid * per_w

        def load_idx(ci, vbuf):
            w = ci // _NB
            b0 = (ci % _NB) * _C
            pltpu.sync_copy(vt_hbm.at[w, pl.ds(b0, _C)], vbuf)

        def start_maps(vbuf, pbuf, sbuf, sem):
            m1 = pltpu.async_copy(pm_hbm.at[vbuf], pbuf, sem)
            m2 = pltpu.async_copy(sm_hbm.at[vbuf], sbuf, sem)
            return m1, m2

        def start_rows(vbuf, pbuf, sbuf, be, bp, bs, sem):
            r1 = pltpu.async_copy(e_hbm.at[vbuf], be, sem)
            r2 = pltpu.async_copy(ep_hbm.at[pbuf], bp, sem)
            r3 = pltpu.async_copy(es_hbm.at[sbuf], bs, sem)
            return r1, r2, r3

        def add_chunk(be, bp, bs, ob):
            def row_body(i, c2):
                for o in (0, 16, 32, 48):
                    ob[i, pl.ds(o, 16)] = (
                        be[i, pl.ds(o, 16)]
                        + bp[i, pl.ds(o, 16)]
                        + bs[i, pl.ds(o, 16)]
                    )
                return c2

            lax.fori_loop(0, _C, row_body, 0)

        def start_wb(ci, ob, sem):
            w = ci // _NB
            b0 = (ci % _NB) * _C
            return pltpu.async_copy(ob, out_hbm.at[w, pl.ds(b0, _C)], sem)

        def wait_wb(ob, sem):
            pltpu.make_async_copy(ob, out_hbm.at[0, pl.ds(0, _C)], sem).wait()

        load_idx(cbase, vi_a)
        m1, m2 = start_maps(vi_a, pi_a, si_a, sem_ia)
        m1.wait()
        m2.wait()

        def pair_body(j, carry):
            c0 = cbase + 2 * j
            c1 = c0 + 1
            c2 = lax.min(c0 + 2, _NCHUNKS - 1)

            r = start_rows(vi_a, pi_a, si_a, be_a, bp_a, bs_a, sem_ra)
            load_idx(c1, vi_b)
            mb = start_maps(vi_b, pi_b, si_b, sem_ib)

            @pl.when(j > 0)
            def _():
                wait_wb(ob_a, sem_wa)

            for d in r:
                d.wait()
            add_chunk(be_a, bp_a, bs_a, ob_a)
            start_wb(c0, ob_a, sem_wa)
            for d in mb:
                d.wait()

            r = start_rows(vi_b, pi_b, si_b, be_b, bp_b, bs_b, sem_rb)
            load_idx(c2, vi_a)
            ma = start_maps(vi_a, pi_a, si_a, sem_ia)

            @pl.when(j > 0)
            def _():
                wait_wb(ob_b, sem_wb)

            for d in r:
                d.wait()
            add_chunk(be_b, bp_b, bs_b, ob_b)
            start_wb(c1, ob_b, sem_wb)
            for d in ma:
                d.wait()
            return carry

        lax.fori_loop(0, pairs, pair_body, 0)
        wait_wb(ob_a, sem_wa)
        wait_wb(ob_b, sem_wb)

    return gather_kernel(vt, pref_map, suff_map, e_pad, ep_pad, es_pad)


def _mlp_tc(h, w0, b0, w1, b1):
    bblk = 1024

    def body(h_ref, w0_ref, b0_ref, w1_ref, b1_ref, o_ref):
        acc = jnp.broadcast_to(b0_ref[...], (bblk, _HID))
        for w in range(_WIN):
            acc = acc + jnp.dot(h_ref[w], w0_ref[w],
                                preferred_element_type=jnp.float32)
        z = jnp.tanh(acc)
        logits = (
            jnp.dot(z, w1_ref[...], preferred_element_type=jnp.float32)
            + b1_ref[...]
        )
        m = jnp.max(logits, axis=1, keepdims=True)
        s = jnp.sum(jnp.exp(logits - m), axis=1, keepdims=True)
        o_ref[...] = logits - (m + jnp.log(s))

    return pl.pallas_call(
        body,
        grid=(_BATCH // bblk,),
        in_specs=[
            pl.BlockSpec((_WIN, bblk, _EMBP), lambda i: (0, i, 0)),
            pl.BlockSpec((_WIN, _EMBP, _HID), lambda i: (0, 0, 0)),
            pl.BlockSpec((1, _HID), lambda i: (0, 0)),
            pl.BlockSpec((_HID, _TAGS), lambda i: (0, 0)),
            pl.BlockSpec((1, _TAGS), lambda i: (0, 0)),
        ],
        out_specs=pl.BlockSpec((bblk, _TAGS), lambda i: (i, 0)),
        out_shape=jax.ShapeDtypeStruct((_BATCH, _TAGS), jnp.float32),
    )(h, w0, b0, w1, b1)


def kernel(v, pref_map, suff_map, E, E_pref, E_suff, W0, b0, W1, b1):
    pad = ((0, 0), (0, _EMBP - _EMB))
    e_pad = jnp.pad(E, pad)
    ep_pad = jnp.pad(E_pref, pad)
    es_pad = jnp.pad(E_suff, pad)
    h = _gather_sum_sc(v.T, pref_map, suff_map, e_pad, ep_pad, es_pad)
    w0p = jnp.pad(W0.reshape(_WIN, _EMB, _HID),
                  ((0, 0), (0, _EMBP - _EMB), (0, 0)))
    return _mlp_tc(h, w0p, b0.reshape(1, _HID), W1, b1.reshape(1, _TAGS))
